# baseline (speedup 1.0000x reference)
import jax
import jax.numpy as jnp
from jax import lax
from jax.experimental import pallas as pl
from jax.experimental.pallas import tpu as pltpu
from jax.experimental.pallas import tpu_sc as plsc

N = 100000
E = 3200000
NS = 16
L = 16

SUB = 80
K = 320
EAS = 8
EPW = E // NS
NCHUNK = EPW // K
RPC = K // SUB
NPAD = 100096
RSUB = NPAD // NS
ZF = RSUB // 2
EAS6 = 6
EAC = K * EAS6
NZA = RSUB // K


def _prep_body(x_ref, A_ref, b1a_ref, xa0_ref, xa1_ref):
    n = jnp.dot(x_ref[...], A_ref[...], preferred_element_type=jnp.float32)
    n = n + b1a_ref[...]
    xa0_ref[...] = n[:, :16]
    xa1_ref[...] = n[:, 16:]


def _sc_run(s, row2, col2, ea_hbm, xa_hbm, B_hbm,
            row_v, col_v, gath, ea_v, B_v, ones_v,
            acc_sp, cnt_sp, lsem, gsem, ssem, with_counts):
    pltpu.sync_copy(B_hbm, B_v)
    b = [B_v[k] for k in range(6)]
    splat_lo = [jnp.full((L,), k, jnp.int32) for k in range(6)]
    splat_hi = [jnp.full((L,), EAS6 + k, jnp.int32) for k in range(6)]

    def lin_issue(ci, p2, p4):
        rbase = s * (EPW // SUB) + ci * RPC
        abase = s * (EPW * EAS6) + ci * EAC
        pltpu.async_copy(row2.at[pl.ds(rbase, RPC)], row_v[p2], lsem[p2])
        pltpu.async_copy(col2.at[pl.ds(rbase, RPC)], col_v[p4], lsem[p2])
        pltpu.async_copy(ea_hbm.at[pl.ds(abase, EAC)],
                         ea_v[p2].at[pl.ds(0, EAC)], lsem[p2])

    def lin_wait(p2, p4):
        pltpu.make_async_copy(row2.at[pl.ds(0, RPC)], row_v[p2], lsem[p2]).wait()
        pltpu.make_async_copy(col2.at[pl.ds(0, RPC)], col_v[p4], lsem[p2]).wait()
        pltpu.make_async_copy(ea_hbm.at[pl.ds(0, EAC)],
                              ea_v[p2].at[pl.ds(0, EAC)], lsem[p2]).wait()

    def gather_issue(p2):
        for j in range(RPC):
            pltpu.async_copy(xa_hbm.at[row_v[p2].at[j]],
                             gath[p2].at[pl.ds(j * SUB, SUB)], gsem[p2])

    def gather_wait(p2):
        for j in range(RPC):
            pltpu.make_async_copy(xa_hbm.at[row_v[p2].at[j]],
                                  gath[p2].at[pl.ds(j * SUB, SUB)],
                                  gsem[p2]).wait()

    def scatter_issue(p2, p4):
        for j in range(RPC):
            pltpu.async_copy(gath[p2].at[pl.ds(j * SUB, SUB)],
                             acc_sp.at[col_v[p4].at[j]], ssem[p2], add=True)
        if with_counts:
            for j in range(RPC):
                pltpu.async_copy(ones_v.at[pl.ds(0, SUB)],
                                 cnt_sp.at[col_v[p4].at[j]], ssem[p2], add=True)

    def scatter_drain(p2, p4):
        for j in range(RPC):
            pltpu.make_async_copy(gath[p2].at[pl.ds(j * SUB, SUB)],
                                  acc_sp.at[col_v[p4].at[j]], ssem[p2]).wait()
        if with_counts:
            for j in range(RPC):
                pltpu.make_async_copy(ones_v.at[pl.ds(0, SUB)],
                                      cnt_sp.at[col_v[p4].at[j]], ssem[p2]).wait()

    def compute(p2):
        gv = gath[p2]
        ev = ea_v[p2]

        def body(t, cc):
            for q in range(4):
                p = t * 4 + q
                r = p * 2
                ea_vec = ev[pl.ds(p * 2 * EAS6, L)]
                e = [ea_vec.at[splat_lo[k]].get(mode="promise_in_bounds")
                     for k in range(6)]
                f = [ea_vec.at[splat_hi[k]].get(mode="promise_in_bounds")
                     for k in range(6)]
                v = (gv[r] + (e[0] * b[0] + e[1] * b[1])
                     + ((e[2] * b[2] + e[3] * b[3])
                        + (e[4] * b[4] + e[5] * b[5])))
                w = (gv[r + 1] + (f[0] * b[0] + f[1] * b[1])
                     + ((f[2] * b[2] + f[3] * b[3])
                        + (f[4] * b[4] + f[5] * b[5])))
                gv[r] = jnp.maximum(v, 0.01 * v)
                gv[r + 1] = jnp.maximum(w, 0.01 * w)
            return cc

        lax.fori_loop(0, K // 8, body, 0)

    lin_issue(0, 0, 0)
    lin_issue(1, 1, 1)
    lin_wait(0, 0)
    gather_issue(0)

    def macro(I, carry):
        for p in range(4):
            i = I * 4 + p
            s2 = p % 2
            s2n = (p + 1) % 2
            s4 = p
            s4n = (p + 1) % 4
            s4nn = (p + 2) % 4
            if p == 0:
                @pl.when(I >= 1)
                def _():
                    scatter_drain(1, 3)
            else:
                scatter_drain(s2n, s4 - 1)
            @pl.when(i <= NCHUNK - 2)
            def _():
                lin_wait(s2n, s4n)
                gather_issue(s2n)
            gather_wait(s2)
            compute(s2)
            scatter_issue(s2, s4)
            @pl.when(i <= NCHUNK - 3)
            def _():
                lin_issue(i + 2, s2, s4nn)
        return carry

    lax.fori_loop(0, NCHUNK // 4, macro, 0)
    scatter_drain(1, 3)


def _edge_sc_body(row2, col2, ea_flat, xa0, xa1, B0, B1,
                  s_out, cnt_out,
                  row_v0, row_v1, col_v0, col_v1, col_v2, col_v3,
                  gath0, gath1, ea_v0, ea_v1, B_v, ones_v, zflat,
                  acc_sp, cnt_sp,
                  lsem0, lsem1, gsem0, gsem1, ssem0, ssem1):
    c = lax.axis_index("c")
    s = lax.axis_index("s")
    row_v = [row_v0, row_v1]
    col_v = [col_v0, col_v1, col_v2, col_v3]
    gath = [gath0, gath1]
    ea_v = [ea_v0, ea_v1]
    lsem = [lsem0, lsem1]
    gsem = [gsem0, gsem1]
    ssem = [ssem0, ssem1]

    zero16 = jnp.zeros((L,), jnp.float32)
    one16 = jnp.ones((L,), jnp.float32)

    def zr(i, cc):
        gath0[i] = zero16
        return cc
    lax.fori_loop(0, K, zr, 0)

    def zf(i, cc):
        zflat[pl.ds(i * L, L)] = zero16
        return cc
    lax.fori_loop(0, (ZF + 8) // L, zf, 0)
    for j in range(SUB // L):
        ones_v[pl.ds(j * L, L)] = one16

    rb = s * RSUB
    for j in range(NZA):
        pltpu.sync_copy(gath0, acc_sp.at[pl.ds(rb + j * K, K)])
    pltpu.sync_copy(gath0.at[pl.ds(0, RSUB - NZA * K)],
                    acc_sp.at[pl.ds(rb + NZA * K, RSUB - NZA * K)])
    for j in range(2):
        pltpu.sync_copy(zflat.at[pl.ds(0, ZF)], cnt_sp.at[pl.ds(rb + j * ZF, ZF)])

    plsc.subcore_barrier()

    @pl.when(c == 0)
    def _():
        _sc_run(s, row2, col2, ea_flat, xa0, B0, row_v, col_v, gath, ea_v,
                B_v, ones_v, acc_sp, cnt_sp, lsem, gsem, ssem,
                with_counts=True)

    @pl.when(c == 1)
    def _():
        _sc_run(s, row2, col2, ea_flat, xa1, B1, row_v, col_v, gath, ea_v,
                B_v, ones_v, acc_sp, cnt_sp, lsem, gsem, ssem,
                with_counts=False)

    plsc.subcore_barrier()

    pltpu.sync_copy(acc_sp.at[pl.ds(rb, RSUB)],
                    s_out.at[pl.ds(c * NPAD + rb, RSUB)])

    @pl.when(c == 0)
    def _():
        pltpu.sync_copy(cnt_sp.at[pl.ds(rb, RSUB)], cnt_out.at[pl.ds(rb, RSUB)])


_edge_sc = pl.kernel(
    _edge_sc_body,
    out_type=(
        jax.ShapeDtypeStruct((2 * NPAD, 16), jnp.float32),
        jax.ShapeDtypeStruct((NPAD,), jnp.float32),
    ),
    mesh=plsc.VectorSubcoreMesh(core_axis_name="c", subcore_axis_name="s"),
    scratch_types=(
        pltpu.VMEM((RPC, SUB), jnp.int32),
        pltpu.VMEM((RPC, SUB), jnp.int32),
        pltpu.VMEM((RPC, SUB), jnp.int32),
        pltpu.VMEM((RPC, SUB), jnp.int32),
        pltpu.VMEM((RPC, SUB), jnp.int32),
        pltpu.VMEM((RPC, SUB), jnp.int32),
        pltpu.VMEM((K, 16), jnp.float32),
        pltpu.VMEM((K, 16), jnp.float32),
        pltpu.VMEM((EAC + 8,), jnp.float32),
        pltpu.VMEM((EAC + 8,), jnp.float32),
        pltpu.VMEM((6, 16), jnp.float32),
        pltpu.VMEM((SUB, ), jnp.float32),
        pltpu.VMEM((ZF + 8,), jnp.float32),
        pltpu.VMEM_SHARED((NPAD, 16), jnp.float32),
        pltpu.VMEM_SHARED((NPAD,), jnp.float32),
        pltpu.SemaphoreType.DMA,
        pltpu.SemaphoreType.DMA,
        pltpu.SemaphoreType.DMA,
        pltpu.SemaphoreType.DMA,
        pltpu.SemaphoreType.DMA,
        pltpu.SemaphoreType.DMA,
    ),
    compiler_params=pltpu.CompilerParams(use_tc_tiling_on_sc=False),
)


def _node_body(x_ref, s0_ref, s1_ref, cnt_ref, bat_ref,
               W2ax_ref, W1b_ref, W2am_ref, b1b_ref, u_ref, W2au_ref,
               b2a_ref, W2b_ref, b2b_ref, out_ref):
    f32 = jnp.float32
    cnt = cnt_ref[...]
    maxc = jnp.maximum(cnt, 1.0)
    inv = 1.0 / maxc
    C = jnp.dot(W1b_ref[...], W2am_ref[...], preferred_element_type=f32)
    dvec = jnp.dot(b1b_ref[...], W2am_ref[...], preferred_element_type=f32)
    ug = jnp.dot(u_ref[...], W2au_ref[...], preferred_element_type=f32)
    Sn = jnp.concatenate([s0_ref[...], s1_ref[...]], axis=1) * inv
    z = jnp.dot(x_ref[...], W2ax_ref[...], preferred_element_type=f32)
    z = z + jnp.dot(Sn, C, preferred_element_type=f32)
    z = z + (cnt * inv) * dvec
    bat = bat_ref[...]
    iota = lax.broadcasted_iota(jnp.int32, (bat.shape[0], 256), 1)
    oh = (bat == iota).astype(f32)
    z = z + jnp.dot(oh, ug, preferred_element_type=f32)
    z = z + b2a_ref[...]
    zz = jnp.maximum(z, 0.01 * z)
    y = jnp.dot(zz, W2b_ref[...], preferred_element_type=f32) + b2b_ref[...]
    out_ref[...] = y


def kernel(x, edge_index, edge_attr, u, batch,
           W1a, b1a, W1b, b1b, W2a, b2a, W2b, b2b):
    f32 = jnp.float32
    A = W1a[:7]
    b1a2 = b1a.reshape(1, 32)

    XB_BLK = 2000
    xa0, xa1 = pl.pallas_call(
        _prep_body,
        grid=(N // XB_BLK,),
        in_specs=[
            pl.BlockSpec((XB_BLK, 7), lambda i: (i, 0)),
            pl.BlockSpec((7, 32), lambda i: (0, 0)),
            pl.BlockSpec((1, 32), lambda i: (0, 0)),
        ],
        out_specs=[
            pl.BlockSpec((XB_BLK, 16), lambda i: (i, 0)),
            pl.BlockSpec((XB_BLK, 16), lambda i: (i, 0)),
        ],
        out_shape=[
            jax.ShapeDtypeStruct((N, 16), f32),
            jax.ShapeDtypeStruct((N, 16), f32),
        ],
    )(x, A, b1a2)

    row2 = edge_index[0].reshape(E // SUB, SUB)
    col2 = edge_index[1].reshape(E // SUB, SUB)
    ea_lin = edge_attr.reshape(E * 6)

    s_flat, cnt_pad = _edge_sc(row2, col2, ea_lin, xa0, xa1,
                               W1a[7:, :16], W1a[7:, 16:])

    s0 = s_flat[:N]
    s1 = s_flat[NPAD:NPAD + N]
    cnt = cnt_pad[:N].reshape(N, 1)
    bat2 = batch.reshape(N, 1)

    NB = 2000
    out = pl.pallas_call(
        _node_body,
        grid=(N // NB,),
        in_specs=[
            pl.BlockSpec((NB, 7), lambda i: (i, 0)),
            pl.BlockSpec((NB, 16), lambda i: (i, 0)),
            pl.BlockSpec((NB, 16), lambda i: (i, 0)),
            pl.BlockSpec((NB, 1), lambda i: (i, 0)),
            pl.BlockSpec((NB, 1), lambda i: (i, 0)),
            pl.BlockSpec((7, 32), lambda i: (0, 0)),
            pl.BlockSpec((32, 32), lambda i: (0, 0)),
            pl.BlockSpec((32, 32), lambda i: (0, 0)),
            pl.BlockSpec((1, 32), lambda i: (0, 0)),
            pl.BlockSpec((256, 64), lambda i: (0, 0)),
            pl.BlockSpec((64, 32), lambda i: (0, 0)),
            pl.BlockSpec((1, 32), lambda i: (0, 0)),
            pl.BlockSpec((32, 7), lambda i: (0, 0)),
            pl.BlockSpec((1, 7), lambda i: (0, 0)),
        ],
        out_specs=pl.BlockSpec((NB, 7), lambda i: (i, 0)),
        out_shape=jax.ShapeDtypeStruct((N, 7), f32),
    )(x, s0, s1, cnt, bat2,
      W2a[:7], W1b, W2a[7:39], b1b.reshape(1, 32), u, W2a[39:],
      b2a.reshape(1, 32), W2b, b2b.reshape(1, 7))
    return out


# speedup vs baseline: 1.0201x; 1.0201x over previous
import jax
import jax.numpy as jnp
from jax import lax
from jax.experimental import pallas as pl
from jax.experimental.pallas import tpu as pltpu
from jax.experimental.pallas import tpu_sc as plsc

N = 100000
E = 3200000
NS = 16
L = 16

SUB = 80
K = 320
EAS = 8
EPW = E // NS
NCHUNK = EPW // K
RPC = K // SUB
NPAD = 100096
RSUB = NPAD // NS
ZF = RSUB // 2
EAS6 = 6
EAC = K * EAS6
NZA = RSUB // K


def _prep_body(x_ref, A_ref, b1a_ref, xa0_ref, xa1_ref):
    n = jnp.dot(x_ref[...], A_ref[...], preferred_element_type=jnp.float32)
    n = n + b1a_ref[...]
    xa0_ref[...] = n[:, :16]
    xa1_ref[...] = n[:, 16:]


def _sc_run(s, row2, col2, ea_hbm, xa_hbm, B_hbm,
            row_v, col_v, gath, ea_v, B_v, ones_v,
            acc_sp, cnt_sp, lsem, gsem, ssem, with_counts):
    pltpu.sync_copy(B_hbm, B_v)
    b = [B_v[k] for k in range(6)]
    splat_lo = [jnp.full((L,), k, jnp.int32) for k in range(6)]
    splat_hi = [jnp.full((L,), EAS6 + k, jnp.int32) for k in range(6)]

    def lin_issue(ci, p2, p4):
        rbase = s * (EPW // SUB) + ci * RPC
        abase = s * (EPW * EAS6) + ci * EAC
        pltpu.async_copy(row2.at[pl.ds(rbase, RPC)], row_v[p2], lsem[p2])
        pltpu.async_copy(col2.at[pl.ds(rbase, RPC)], col_v[p4], lsem[p2])
        pltpu.async_copy(ea_hbm.at[pl.ds(abase, EAC)],
                         ea_v[p2].at[pl.ds(0, EAC)], lsem[p2])

    def lin_wait(p2, p4):
        pltpu.make_async_copy(row2.at[pl.ds(0, RPC)], row_v[p2], lsem[p2]).wait()
        pltpu.make_async_copy(col2.at[pl.ds(0, RPC)], col_v[p4], lsem[p2]).wait()
        pltpu.make_async_copy(ea_hbm.at[pl.ds(0, EAC)],
                              ea_v[p2].at[pl.ds(0, EAC)], lsem[p2]).wait()

    def gather_issue(p2):
        for j in range(RPC):
            pltpu.async_copy(xa_hbm.at[row_v[p2].at[j]],
                             gath[p2].at[pl.ds(j * SUB, SUB)], gsem[p2])

    def gather_wait(p2):
        for j in range(RPC):
            pltpu.make_async_copy(xa_hbm.at[row_v[p2].at[j]],
                                  gath[p2].at[pl.ds(j * SUB, SUB)],
                                  gsem[p2]).wait()

    def scatter_issue(p2, p4):
        for j in range(RPC):
            pltpu.async_copy(gath[p2].at[pl.ds(j * SUB, SUB)],
                             acc_sp.at[col_v[p4].at[j]], ssem[p2], add=True)
        if with_counts:
            for j in range(RPC):
                pltpu.async_copy(ones_v.at[pl.ds(0, SUB)],
                                 cnt_sp.at[col_v[p4].at[j]], ssem[p2], add=True)

    def scatter_drain(p2, p4):
        for j in range(RPC):
            pltpu.make_async_copy(gath[p2].at[pl.ds(j * SUB, SUB)],
                                  acc_sp.at[col_v[p4].at[j]], ssem[p2]).wait()
        if with_counts:
            for j in range(RPC):
                pltpu.make_async_copy(ones_v.at[pl.ds(0, SUB)],
                                      cnt_sp.at[col_v[p4].at[j]], ssem[p2]).wait()

    def compute(p2):
        gv = gath[p2]
        ev = ea_v[p2]

        def body(t, cc):
            for q in range(2):
                p = t * 2 + q
                r = p * 2
                ea_vec = ev[pl.ds(p * 2 * EAS6, L)]
                e = [ea_vec.at[splat_lo[k]].get(mode="promise_in_bounds")
                     for k in range(6)]
                f = [ea_vec.at[splat_hi[k]].get(mode="promise_in_bounds")
                     for k in range(6)]
                v = (gv[r] + (e[0] * b[0] + e[1] * b[1])
                     + ((e[2] * b[2] + e[3] * b[3])
                        + (e[4] * b[4] + e[5] * b[5])))
                w = (gv[r + 1] + (f[0] * b[0] + f[1] * b[1])
                     + ((f[2] * b[2] + f[3] * b[3])
                        + (f[4] * b[4] + f[5] * b[5])))
                gv[r] = jnp.maximum(v, 0.01 * v)
                gv[r + 1] = jnp.maximum(w, 0.01 * w)
            return cc

        lax.fori_loop(0, K // 4, body, 0)

    lin_issue(0, 0, 0)
    lin_issue(1, 1, 1)
    lin_wait(0, 0)
    gather_issue(0)

    def macro(I, carry):
        for p in range(4):
            i = I * 4 + p
            s2 = p % 2
            s2n = (p + 1) % 2
            s4 = p
            s4n = (p + 1) % 4
            s4nn = (p + 2) % 4
            if p == 0:
                @pl.when(I >= 1)
                def _():
                    scatter_drain(1, 3)
            else:
                scatter_drain(s2n, s4 - 1)
            @pl.when(i <= NCHUNK - 2)
            def _():
                lin_wait(s2n, s4n)
                gather_issue(s2n)
            gather_wait(s2)
            compute(s2)
            scatter_issue(s2, s4)
            @pl.when(i <= NCHUNK - 3)
            def _():
                lin_issue(i + 2, s2, s4nn)
        return carry

    lax.fori_loop(0, NCHUNK // 4, macro, 0)
    scatter_drain(1, 3)


def _edge_sc_body(row2, col2, ea_flat, xa0, xa1, B0, B1,
                  s_out, cnt_out,
                  row_v0, row_v1, col_v0, col_v1, col_v2, col_v3,
                  gath0, gath1, ea_v0, ea_v1, B_v, ones_v, zflat,
                  acc_sp, cnt_sp,
                  lsem0, lsem1, gsem0, gsem1, ssem0, ssem1):
    c = lax.axis_index("c")
    s = lax.axis_index("s")
    row_v = [row_v0, row_v1]
    col_v = [col_v0, col_v1, col_v2, col_v3]
    gath = [gath0, gath1]
    ea_v = [ea_v0, ea_v1]
    lsem = [lsem0, lsem1]
    gsem = [gsem0, gsem1]
    ssem = [ssem0, ssem1]

    zero16 = jnp.zeros((L,), jnp.float32)
    one16 = jnp.ones((L,), jnp.float32)

    def zr(i, cc):
        gath0[i] = zero16
        return cc
    lax.fori_loop(0, K, zr, 0)

    def zf(i, cc):
        zflat[pl.ds(i * L, L)] = zero16
        return cc
    lax.fori_loop(0, (ZF + 8) // L, zf, 0)
    for j in range(SUB // L):
        ones_v[pl.ds(j * L, L)] = one16

    rb = s * RSUB
    for j in range(NZA):
        pltpu.sync_copy(gath0, acc_sp.at[pl.ds(rb + j * K, K)])
    pltpu.sync_copy(gath0.at[pl.ds(0, RSUB - NZA * K)],
                    acc_sp.at[pl.ds(rb + NZA * K, RSUB - NZA * K)])
    for j in range(2):
        pltpu.sync_copy(zflat.at[pl.ds(0, ZF)], cnt_sp.at[pl.ds(rb + j * ZF, ZF)])

    plsc.subcore_barrier()

    @pl.when(c == 0)
    def _():
        _sc_run(s, row2, col2, ea_flat, xa0, B0, row_v, col_v, gath, ea_v,
                B_v, ones_v, acc_sp, cnt_sp, lsem, gsem, ssem,
                with_counts=True)

    @pl.when(c == 1)
    def _():
        _sc_run(s, row2, col2, ea_flat, xa1, B1, row_v, col_v, gath, ea_v,
                B_v, ones_v, acc_sp, cnt_sp, lsem, gsem, ssem,
                with_counts=False)

    plsc.subcore_barrier()

    pltpu.sync_copy(acc_sp.at[pl.ds(rb, RSUB)],
                    s_out.at[pl.ds(c * NPAD + rb, RSUB)])

    @pl.when(c == 0)
    def _():
        pltpu.sync_copy(cnt_sp.at[pl.ds(rb, RSUB)], cnt_out.at[pl.ds(rb, RSUB)])


_edge_sc = pl.kernel(
    _edge_sc_body,
    out_type=(
        jax.ShapeDtypeStruct((2 * NPAD, 16), jnp.float32),
        jax.ShapeDtypeStruct((NPAD,), jnp.float32),
    ),
    mesh=plsc.VectorSubcoreMesh(core_axis_name="c", subcore_axis_name="s"),
    scratch_types=(
        pltpu.VMEM((RPC, SUB), jnp.int32),
        pltpu.VMEM((RPC, SUB), jnp.int32),
        pltpu.VMEM((RPC, SUB), jnp.int32),
        pltpu.VMEM((RPC, SUB), jnp.int32),
        pltpu.VMEM((RPC, SUB), jnp.int32),
        pltpu.VMEM((RPC, SUB), jnp.int32),
        pltpu.VMEM((K, 16), jnp.float32),
        pltpu.VMEM((K, 16), jnp.float32),
        pltpu.VMEM((EAC + 8,), jnp.float32),
        pltpu.VMEM((EAC + 8,), jnp.float32),
        pltpu.VMEM((6, 16), jnp.float32),
        pltpu.VMEM((SUB, ), jnp.float32),
        pltpu.VMEM((ZF + 8,), jnp.float32),
        pltpu.VMEM_SHARED((NPAD, 16), jnp.float32),
        pltpu.VMEM_SHARED((NPAD,), jnp.float32),
        pltpu.SemaphoreType.DMA,
        pltpu.SemaphoreType.DMA,
        pltpu.SemaphoreType.DMA,
        pltpu.SemaphoreType.DMA,
        pltpu.SemaphoreType.DMA,
        pltpu.SemaphoreType.DMA,
    ),
    compiler_params=pltpu.CompilerParams(use_tc_tiling_on_sc=False),
)


def _node_body(x_ref, s0_ref, s1_ref, cnt_ref, bat_ref,
               W2ax_ref, W1b_ref, W2am_ref, b1b_ref, u_ref, W2au_ref,
               b2a_ref, W2b_ref, b2b_ref, out_ref):
    f32 = jnp.float32
    cnt = cnt_ref[...]
    maxc = jnp.maximum(cnt, 1.0)
    inv = 1.0 / maxc
    C = jnp.dot(W1b_ref[...], W2am_ref[...], preferred_element_type=f32)
    dvec = jnp.dot(b1b_ref[...], W2am_ref[...], preferred_element_type=f32)
    ug = jnp.dot(u_ref[...], W2au_ref[...], preferred_element_type=f32)
    Sn = jnp.concatenate([s0_ref[...], s1_ref[...]], axis=1) * inv
    z = jnp.dot(x_ref[...], W2ax_ref[...], preferred_element_type=f32)
    z = z + jnp.dot(Sn, C, preferred_element_type=f32)
    z = z + (cnt * inv) * dvec
    bat = bat_ref[...]
    iota = lax.broadcasted_iota(jnp.int32, (bat.shape[0], 256), 1)
    oh = (bat == iota).astype(f32)
    z = z + jnp.dot(oh, ug, preferred_element_type=f32)
    z = z + b2a_ref[...]
    zz = jnp.maximum(z, 0.01 * z)
    y = jnp.dot(zz, W2b_ref[...], preferred_element_type=f32) + b2b_ref[...]
    out_ref[...] = y


def kernel(x, edge_index, edge_attr, u, batch,
           W1a, b1a, W1b, b1b, W2a, b2a, W2b, b2b):
    f32 = jnp.float32
    A = W1a[:7]
    b1a2 = b1a.reshape(1, 32)

    XB_BLK = 2000
    xa0, xa1 = pl.pallas_call(
        _prep_body,
        grid=(N // XB_BLK,),
        in_specs=[
            pl.BlockSpec((XB_BLK, 7), lambda i: (i, 0)),
            pl.BlockSpec((7, 32), lambda i: (0, 0)),
            pl.BlockSpec((1, 32), lambda i: (0, 0)),
        ],
        out_specs=[
            pl.BlockSpec((XB_BLK, 16), lambda i: (i, 0)),
            pl.BlockSpec((XB_BLK, 16), lambda i: (i, 0)),
        ],
        out_shape=[
            jax.ShapeDtypeStruct((N, 16), f32),
            jax.ShapeDtypeStruct((N, 16), f32),
        ],
    )(x, A, b1a2)

    row2 = edge_index[0].reshape(E // SUB, SUB)
    col2 = edge_index[1].reshape(E // SUB, SUB)
    ea_lin = edge_attr.reshape(E * 6)

    s_flat, cnt_pad = _edge_sc(row2, col2, ea_lin, xa0, xa1,
                               W1a[7:, :16], W1a[7:, 16:])

    s0 = s_flat[:N]
    s1 = s_flat[NPAD:NPAD + N]
    cnt = cnt_pad[:N].reshape(N, 1)
    bat2 = batch.reshape(N, 1)

    NB = 2000
    out = pl.pallas_call(
        _node_body,
        grid=(N // NB,),
        in_specs=[
            pl.BlockSpec((NB, 7), lambda i: (i, 0)),
            pl.BlockSpec((NB, 16), lambda i: (i, 0)),
            pl.BlockSpec((NB, 16), lambda i: (i, 0)),
            pl.BlockSpec((NB, 1), lambda i: (i, 0)),
            pl.BlockSpec((NB, 1), lambda i: (i, 0)),
            pl.BlockSpec((7, 32), lambda i: (0, 0)),
            pl.BlockSpec((32, 32), lambda i: (0, 0)),
            pl.BlockSpec((32, 32), lambda i: (0, 0)),
            pl.BlockSpec((1, 32), lambda i: (0, 0)),
            pl.BlockSpec((256, 64), lambda i: (0, 0)),
            pl.BlockSpec((64, 32), lambda i: (0, 0)),
            pl.BlockSpec((1, 32), lambda i: (0, 0)),
            pl.BlockSpec((32, 7), lambda i: (0, 0)),
            pl.BlockSpec((1, 7), lambda i: (0, 0)),
        ],
        out_specs=pl.BlockSpec((NB, 7), lambda i: (i, 0)),
        out_shape=jax.ShapeDtypeStruct((N, 7), f32),
    )(x, s0, s1, cnt, bat2,
      W2a[:7], W1b, W2a[7:39], b1b.reshape(1, 32), u, W2a[39:],
      b2a.reshape(1, 32), W2b, b2b.reshape(1, 7))
    return out


# speedup vs baseline: 1.0462x; 1.0256x over previous
import jax
import jax.numpy as jnp
from jax import lax
from jax.experimental import pallas as pl
from jax.experimental.pallas import tpu as pltpu
from jax.experimental.pallas import tpu_sc as plsc

N = 100000
E = 3200000
NS = 16
L = 16

SUB = 80
K = 320
EAS = 8
EPW = E // NS
NCHUNK = EPW // K
RPC = K // SUB
NPAD = 100096
RSUB = NPAD // NS
ZF = RSUB // 2
EAS6 = 6
EAC = K * EAS6
NZA = RSUB // K


def _prep_body(x_ref, A_ref, b1a_ref, xa0_ref, xa1_ref):
    n = jnp.dot(x_ref[...], A_ref[...], preferred_element_type=jnp.float32)
    n = n + b1a_ref[...]
    xa0_ref[...] = n[:, :16]
    xa1_ref[...] = n[:, 16:]


def _sc_run(s, row2, col2, ea_hbm, xa_hbm, B_hbm,
            row_v, col_v, gath, ea_v, B_v, ones_v,
            acc_sp, cnt_sp, lsem, gsem, ssem, with_counts):
    pltpu.sync_copy(B_hbm, B_v)
    b = [B_v[k] for k in range(6)]
    splat_lo = [jnp.full((L,), k, jnp.int32) for k in range(6)]
    splat_hi = [jnp.full((L,), EAS6 + k, jnp.int32) for k in range(6)]

    def lin_issue(ci, p2, p4):
        rbase = s * (EPW // SUB) + ci * RPC
        abase = s * (EPW * EAS6) + ci * EAC
        pltpu.async_copy(row2.at[pl.ds(rbase, RPC)], row_v[p2], lsem[p2])
        pltpu.async_copy(col2.at[pl.ds(rbase, RPC)], col_v[p4], lsem[p2])
        pltpu.async_copy(ea_hbm.at[pl.ds(abase, EAC)],
                         ea_v[p2].at[pl.ds(0, EAC)], lsem[p2])

    def lin_wait(p2, p4):
        pltpu.make_async_copy(row2.at[pl.ds(0, RPC)], row_v[p2], lsem[p2]).wait()
        pltpu.make_async_copy(col2.at[pl.ds(0, RPC)], col_v[p4], lsem[p2]).wait()
        pltpu.make_async_copy(ea_hbm.at[pl.ds(0, EAC)],
                              ea_v[p2].at[pl.ds(0, EAC)], lsem[p2]).wait()

    def gather_issue(p2):
        for j in range(RPC):
            pltpu.async_copy(xa_hbm.at[row_v[p2].at[j]],
                             gath[p2].at[pl.ds(j * SUB, SUB)], gsem[p2])

    def gather_wait(p2):
        for j in range(RPC):
            pltpu.make_async_copy(xa_hbm.at[row_v[p2].at[j]],
                                  gath[p2].at[pl.ds(j * SUB, SUB)],
                                  gsem[p2]).wait()

    def scatter_issue(p2, p4):
        for j in range(RPC):
            pltpu.async_copy(gath[p2].at[pl.ds(j * SUB, SUB)],
                             acc_sp.at[col_v[p4].at[j]], ssem[p2], add=True)
        if with_counts:
            for j in range(RPC):
                pltpu.async_copy(ones_v.at[pl.ds(0, SUB)],
                                 cnt_sp.at[col_v[p4].at[j]], ssem[p2], add=True)

    def scatter_drain(p2, p4):
        for j in range(RPC):
            pltpu.make_async_copy(gath[p2].at[pl.ds(j * SUB, SUB)],
                                  acc_sp.at[col_v[p4].at[j]], ssem[p2]).wait()
        if with_counts:
            for j in range(RPC):
                pltpu.make_async_copy(ones_v.at[pl.ds(0, SUB)],
                                      cnt_sp.at[col_v[p4].at[j]], ssem[p2]).wait()

    def compute(p2):
        gv = gath[p2]
        ev = ea_v[p2]

        def body(t, cc):
            for q in range(4):
                p = t * 4 + q
                r = p * 2
                ea_vec = ev[pl.ds(p * 2 * EAS6, L)]
                v = gv[r]
                w = gv[r + 1]
                for k in range(6):
                    ek = ea_vec.at[splat_lo[k]].get(mode="promise_in_bounds")
                    fk = ea_vec.at[splat_hi[k]].get(mode="promise_in_bounds")
                    v = v + ek * b[k]
                    w = w + fk * b[k]
                gv[r] = jnp.maximum(v, 0.01 * v)
                gv[r + 1] = jnp.maximum(w, 0.01 * w)
            return cc

        lax.fori_loop(0, K // 8, body, 0)

    lin_issue(0, 0, 0)
    lin_issue(1, 1, 1)
    lin_wait(0, 0)
    gather_issue(0)

    def macro(I, carry):
        for p in range(4):
            i = I * 4 + p
            s2 = p % 2
            s2n = (p + 1) % 2
            s4 = p
            s4n = (p + 1) % 4
            s4nn = (p + 2) % 4
            if p == 0:
                @pl.when(I >= 1)
                def _():
                    scatter_drain(1, 3)
            else:
                scatter_drain(s2n, s4 - 1)
            @pl.when(i <= NCHUNK - 2)
            def _():
                lin_wait(s2n, s4n)
                gather_issue(s2n)
            gather_wait(s2)
            compute(s2)
            scatter_issue(s2, s4)
            @pl.when(i <= NCHUNK - 3)
            def _():
                lin_issue(i + 2, s2, s4nn)
        return carry

    lax.fori_loop(0, NCHUNK // 4, macro, 0)
    scatter_drain(1, 3)


def _edge_sc_body(row2, col2, ea_flat, xa0, xa1, B0, B1,
                  s_out, cnt_out,
                  row_v0, row_v1, col_v0, col_v1, col_v2, col_v3,
                  gath0, gath1, ea_v0, ea_v1, B_v, ones_v, zflat,
                  acc_sp, cnt_sp,
                  lsem0, lsem1, gsem0, gsem1, ssem0, ssem1):
    c = lax.axis_index("c")
    s = lax.axis_index("s")
    row_v = [row_v0, row_v1]
    col_v = [col_v0, col_v1, col_v2, col_v3]
    gath = [gath0, gath1]
    ea_v = [ea_v0, ea_v1]
    lsem = [lsem0, lsem1]
    gsem = [gsem0, gsem1]
    ssem = [ssem0, ssem1]

    zero16 = jnp.zeros((L,), jnp.float32)
    one16 = jnp.ones((L,), jnp.float32)

    def zr(i, cc):
        gath0[i] = zero16
        return cc
    lax.fori_loop(0, K, zr, 0)

    def zf(i, cc):
        zflat[pl.ds(i * L, L)] = zero16
        return cc
    lax.fori_loop(0, (ZF + 8) // L, zf, 0)
    for j in range(SUB // L):
        ones_v[pl.ds(j * L, L)] = one16

    rb = s * RSUB
    for j in range(NZA):
        pltpu.sync_copy(gath0, acc_sp.at[pl.ds(rb + j * K, K)])
    pltpu.sync_copy(gath0.at[pl.ds(0, RSUB - NZA * K)],
                    acc_sp.at[pl.ds(rb + NZA * K, RSUB - NZA * K)])
    for j in range(2):
        pltpu.sync_copy(zflat.at[pl.ds(0, ZF)], cnt_sp.at[pl.ds(rb + j * ZF, ZF)])

    plsc.subcore_barrier()

    @pl.when(c == 0)
    def _():
        _sc_run(s, row2, col2, ea_flat, xa0, B0, row_v, col_v, gath, ea_v,
                B_v, ones_v, acc_sp, cnt_sp, lsem, gsem, ssem,
                with_counts=True)

    @pl.when(c == 1)
    def _():
        _sc_run(s, row2, col2, ea_flat, xa1, B1, row_v, col_v, gath, ea_v,
                B_v, ones_v, acc_sp, cnt_sp, lsem, gsem, ssem,
                with_counts=False)

    plsc.subcore_barrier()

    pltpu.sync_copy(acc_sp.at[pl.ds(rb, RSUB)],
                    s_out.at[pl.ds(c * NPAD + rb, RSUB)])

    @pl.when(c == 0)
    def _():
        pltpu.sync_copy(cnt_sp.at[pl.ds(rb, RSUB)], cnt_out.at[pl.ds(rb, RSUB)])


_edge_sc = pl.kernel(
    _edge_sc_body,
    out_type=(
        jax.ShapeDtypeStruct((2 * NPAD, 16), jnp.float32),
        jax.ShapeDtypeStruct((NPAD,), jnp.float32),
    ),
    mesh=plsc.VectorSubcoreMesh(core_axis_name="c", subcore_axis_name="s"),
    scratch_types=(
        pltpu.VMEM((RPC, SUB), jnp.int32),
        pltpu.VMEM((RPC, SUB), jnp.int32),
        pltpu.VMEM((RPC, SUB), jnp.int32),
        pltpu.VMEM((RPC, SUB), jnp.int32),
        pltpu.VMEM((RPC, SUB), jnp.int32),
        pltpu.VMEM((RPC, SUB), jnp.int32),
        pltpu.VMEM((K, 16), jnp.float32),
        pltpu.VMEM((K, 16), jnp.float32),
        pltpu.VMEM((EAC + 8,), jnp.float32),
        pltpu.VMEM((EAC + 8,), jnp.float32),
        pltpu.VMEM((6, 16), jnp.float32),
        pltpu.VMEM((SUB, ), jnp.float32),
        pltpu.VMEM((ZF + 8,), jnp.float32),
        pltpu.VMEM_SHARED((NPAD, 16), jnp.float32),
        pltpu.VMEM_SHARED((NPAD,), jnp.float32),
        pltpu.SemaphoreType.DMA,
        pltpu.SemaphoreType.DMA,
        pltpu.SemaphoreType.DMA,
        pltpu.SemaphoreType.DMA,
        pltpu.SemaphoreType.DMA,
        pltpu.SemaphoreType.DMA,
    ),
    compiler_params=pltpu.CompilerParams(use_tc_tiling_on_sc=False),
)


def _node_body(x_ref, s0_ref, s1_ref, cnt_ref, bat_ref,
               W2ax_ref, W1b_ref, W2am_ref, b1b_ref, u_ref, W2au_ref,
               b2a_ref, W2b_ref, b2b_ref, out_ref):
    f32 = jnp.float32
    cnt = cnt_ref[...]
    maxc = jnp.maximum(cnt, 1.0)
    inv = 1.0 / maxc
    C = jnp.dot(W1b_ref[...], W2am_ref[...], preferred_element_type=f32)
    dvec = jnp.dot(b1b_ref[...], W2am_ref[...], preferred_element_type=f32)
    ug = jnp.dot(u_ref[...], W2au_ref[...], preferred_element_type=f32)
    Sn = jnp.concatenate([s0_ref[...], s1_ref[...]], axis=1) * inv
    z = jnp.dot(x_ref[...], W2ax_ref[...], preferred_element_type=f32)
    z = z + jnp.dot(Sn, C, preferred_element_type=f32)
    z = z + (cnt * inv) * dvec
    bat = bat_ref[...]
    iota = lax.broadcasted_iota(jnp.int32, (bat.shape[0], 256), 1)
    oh = (bat == iota).astype(f32)
    z = z + jnp.dot(oh, ug, preferred_element_type=f32)
    z = z + b2a_ref[...]
    zz = jnp.maximum(z, 0.01 * z)
    y = jnp.dot(zz, W2b_ref[...], preferred_element_type=f32) + b2b_ref[...]
    out_ref[...] = y


def kernel(x, edge_index, edge_attr, u, batch,
           W1a, b1a, W1b, b1b, W2a, b2a, W2b, b2b):
    f32 = jnp.float32
    A = W1a[:7]
    b1a2 = b1a.reshape(1, 32)

    XB_BLK = 2000
    xa0, xa1 = pl.pallas_call(
        _prep_body,
        grid=(N // XB_BLK,),
        in_specs=[
            pl.BlockSpec((XB_BLK, 7), lambda i: (i, 0)),
            pl.BlockSpec((7, 32), lambda i: (0, 0)),
            pl.BlockSpec((1, 32), lambda i: (0, 0)),
        ],
        out_specs=[
            pl.BlockSpec((XB_BLK, 16), lambda i: (i, 0)),
            pl.BlockSpec((XB_BLK, 16), lambda i: (i, 0)),
        ],
        out_shape=[
            jax.ShapeDtypeStruct((N, 16), f32),
            jax.ShapeDtypeStruct((N, 16), f32),
        ],
    )(x, A, b1a2)

    row2 = edge_index[0].reshape(E // SUB, SUB)
    col2 = edge_index[1].reshape(E // SUB, SUB)
    ea_lin = edge_attr.reshape(E * 6)

    s_flat, cnt_pad = _edge_sc(row2, col2, ea_lin, xa0, xa1,
                               W1a[7:, :16], W1a[7:, 16:])

    s0 = s_flat[:N]
    s1 = s_flat[NPAD:NPAD + N]
    cnt = cnt_pad[:N].reshape(N, 1)
    bat2 = batch.reshape(N, 1)

    NB = 2000
    out = pl.pallas_call(
        _node_body,
        grid=(N // NB,),
        in_specs=[
            pl.BlockSpec((NB, 7), lambda i: (i, 0)),
            pl.BlockSpec((NB, 16), lambda i: (i, 0)),
            pl.BlockSpec((NB, 16), lambda i: (i, 0)),
            pl.BlockSpec((NB, 1), lambda i: (i, 0)),
            pl.BlockSpec((NB, 1), lambda i: (i, 0)),
            pl.BlockSpec((7, 32), lambda i: (0, 0)),
            pl.BlockSpec((32, 32), lambda i: (0, 0)),
            pl.BlockSpec((32, 32), lambda i: (0, 0)),
            pl.BlockSpec((1, 32), lambda i: (0, 0)),
            pl.BlockSpec((256, 64), lambda i: (0, 0)),
            pl.BlockSpec((64, 32), lambda i: (0, 0)),
            pl.BlockSpec((1, 32), lambda i: (0, 0)),
            pl.BlockSpec((32, 7), lambda i: (0, 0)),
            pl.BlockSpec((1, 7), lambda i: (0, 0)),
        ],
        out_specs=pl.BlockSpec((NB, 7), lambda i: (i, 0)),
        out_shape=jax.ShapeDtypeStruct((N, 7), f32),
    )(x, s0, s1, cnt, bat2,
      W2a[:7], W1b, W2a[7:39], b1b.reshape(1, 32), u, W2a[39:],
      b2a.reshape(1, 32), W2b, b2b.reshape(1, 7))
    return out


# speedup vs baseline: 1.0690x; 1.0217x over previous
import jax
import jax.numpy as jnp
from jax import lax
from jax.experimental import pallas as pl
from jax.experimental.pallas import tpu as pltpu
from jax.experimental.pallas import tpu_sc as plsc

N = 100000
E = 3200000
NS = 16
L = 16

SUB = 80
K = 320
EAS = 8
EPW = E // NS
NCHUNK = EPW // K
RPC = K // SUB
NPAD = 100096
RSUB = NPAD // NS
ZF = RSUB // 2
EAS6 = 6
EAC = K * EAS6
NZA = RSUB // K


def _prep_body(x_ref, A_ref, b1a_ref, xa0_ref, xa1_ref):
    n = jnp.dot(x_ref[...], A_ref[...], preferred_element_type=jnp.float32)
    n = n + b1a_ref[...]
    xa0_ref[...] = n[:, :16]
    xa1_ref[...] = n[:, 16:]


def _sc_run(s, rc2, ea_hbm, xa_hbm, B_hbm,
            row_v, col_v, gath, ea_v, B_v, ones_v,
            acc_sp, cnt_sp, lsem, gsem, ssem, with_counts):
    pltpu.sync_copy(B_hbm, B_v)
    b = [B_v[k] for k in range(6)]
    splat_lo = [jnp.full((L,), k, jnp.int32) for k in range(6)]
    splat_hi = [jnp.full((L,), EAS6 + k, jnp.int32) for k in range(6)]

    def lin_issue(ci, p2, p4):
        rbase = s * (EPW // SUB) + ci * RPC
        abase = s * (EPW * EAS6) + ci * EAC
        pltpu.async_copy(rc2.at[pl.ds(rbase, RPC)], row_v[p2], lsem[p2])
        pltpu.async_copy(rc2.at[pl.ds(E // SUB + rbase, RPC)], col_v[p4],
                         lsem[p2])
        pltpu.async_copy(ea_hbm.at[pl.ds(abase, EAC)],
                         ea_v[p2].at[pl.ds(0, EAC)], lsem[p2])

    def lin_wait(p2, p4):
        pltpu.make_async_copy(rc2.at[pl.ds(0, RPC)], row_v[p2], lsem[p2]).wait()
        pltpu.make_async_copy(rc2.at[pl.ds(0, RPC)], col_v[p4], lsem[p2]).wait()
        pltpu.make_async_copy(ea_hbm.at[pl.ds(0, EAC)],
                              ea_v[p2].at[pl.ds(0, EAC)], lsem[p2]).wait()

    def gather_issue(p2):
        for j in range(RPC):
            pltpu.async_copy(xa_hbm.at[row_v[p2].at[j]],
                             gath[p2].at[pl.ds(j * SUB, SUB)], gsem[p2])

    def gather_wait(p2):
        for j in range(RPC):
            pltpu.make_async_copy(xa_hbm.at[row_v[p2].at[j]],
                                  gath[p2].at[pl.ds(j * SUB, SUB)],
                                  gsem[p2]).wait()

    def scatter_issue(p2, p4):
        for j in range(RPC):
            pltpu.async_copy(gath[p2].at[pl.ds(j * SUB, SUB)],
                             acc_sp.at[col_v[p4].at[j]], ssem[p2], add=True)
        if with_counts:
            for j in range(RPC):
                pltpu.async_copy(ones_v.at[pl.ds(0, SUB)],
                                 cnt_sp.at[col_v[p4].at[j]], ssem[p2], add=True)

    def scatter_drain(p2, p4):
        for j in range(RPC):
            pltpu.make_async_copy(gath[p2].at[pl.ds(j * SUB, SUB)],
                                  acc_sp.at[col_v[p4].at[j]], ssem[p2]).wait()
        if with_counts:
            for j in range(RPC):
                pltpu.make_async_copy(ones_v.at[pl.ds(0, SUB)],
                                      cnt_sp.at[col_v[p4].at[j]], ssem[p2]).wait()

    def compute(p2):
        gv = gath[p2]
        ev = ea_v[p2]

        def body(t, cc):
            for q in range(2):
                p = t * 2 + q
                r = p * 2
                ea_vec = ev[pl.ds(p * 2 * EAS6, L)]
                v = gv[r]
                w = gv[r + 1]
                for k in range(6):
                    ek = ea_vec.at[splat_lo[k]].get(mode="promise_in_bounds")
                    fk = ea_vec.at[splat_hi[k]].get(mode="promise_in_bounds")
                    v = v + ek * b[k]
                    w = w + fk * b[k]
                gv[r] = jnp.maximum(v, 0.01 * v)
                gv[r + 1] = jnp.maximum(w, 0.01 * w)
            return cc

        lax.fori_loop(0, K // 4, body, 0)

    lin_issue(0, 0, 0)
    lin_issue(1, 1, 1)
    lin_wait(0, 0)
    gather_issue(0)

    def macro(I, carry):
        for p in range(4):
            i = I * 4 + p
            s2 = p % 2
            s2n = (p + 1) % 2
            s4 = p
            s4n = (p + 1) % 4
            s4nn = (p + 2) % 4
            if p == 0:
                @pl.when(I >= 1)
                def _():
                    scatter_drain(1, 3)
            else:
                scatter_drain(s2n, s4 - 1)
            @pl.when(i <= NCHUNK - 2)
            def _():
                lin_wait(s2n, s4n)
                gather_issue(s2n)
            gather_wait(s2)
            compute(s2)
            scatter_issue(s2, s4)
            @pl.when(i <= NCHUNK - 3)
            def _():
                lin_issue(i + 2, s2, s4nn)
        return carry

    lax.fori_loop(0, NCHUNK // 4, macro, 0)
    scatter_drain(1, 3)


def _edge_sc_body(rc2, ea_flat, xa0, xa1, B0, B1,
                  s_out, cnt_out,
                  row_v0, row_v1, col_v0, col_v1, col_v2, col_v3,
                  gath0, gath1, ea_v0, ea_v1, B_v, ones_v, zflat,
                  acc_sp, cnt_sp,
                  lsem0, lsem1, gsem0, gsem1, ssem0, ssem1):
    c = lax.axis_index("c")
    s = lax.axis_index("s")
    row_v = [row_v0, row_v1]
    col_v = [col_v0, col_v1, col_v2, col_v3]
    gath = [gath0, gath1]
    ea_v = [ea_v0, ea_v1]
    lsem = [lsem0, lsem1]
    gsem = [gsem0, gsem1]
    ssem = [ssem0, ssem1]

    zero16 = jnp.zeros((L,), jnp.float32)
    one16 = jnp.ones((L,), jnp.float32)

    def zr(i, cc):
        gath0[i] = zero16
        return cc
    lax.fori_loop(0, K, zr, 0)

    def zf(i, cc):
        zflat[pl.ds(i * L, L)] = zero16
        return cc
    lax.fori_loop(0, (ZF + 8) // L, zf, 0)
    for j in range(SUB // L):
        ones_v[pl.ds(j * L, L)] = one16

    rb = s * RSUB
    for j in range(NZA):
        pltpu.sync_copy(gath0, acc_sp.at[pl.ds(rb + j * K, K)])
    pltpu.sync_copy(gath0.at[pl.ds(0, RSUB - NZA * K)],
                    acc_sp.at[pl.ds(rb + NZA * K, RSUB - NZA * K)])
    for j in range(2):
        pltpu.sync_copy(zflat.at[pl.ds(0, ZF)], cnt_sp.at[pl.ds(rb + j * ZF, ZF)])

    plsc.subcore_barrier()

    @pl.when(c == 0)
    def _():
        _sc_run(s, rc2, ea_flat, xa0, B0, row_v, col_v, gath, ea_v,
                B_v, ones_v, acc_sp, cnt_sp, lsem, gsem, ssem,
                with_counts=True)

    @pl.when(c == 1)
    def _():
        _sc_run(s, rc2, ea_flat, xa1, B1, row_v, col_v, gath, ea_v,
                B_v, ones_v, acc_sp, cnt_sp, lsem, gsem, ssem,
                with_counts=False)

    plsc.subcore_barrier()

    pltpu.sync_copy(acc_sp.at[pl.ds(rb, RSUB)],
                    s_out.at[pl.ds(c * NPAD + rb, RSUB)])

    @pl.when(c == 0)
    def _():
        pltpu.sync_copy(cnt_sp.at[pl.ds(rb, RSUB)], cnt_out.at[pl.ds(rb, RSUB)])


_edge_sc = pl.kernel(
    _edge_sc_body,
    out_type=(
        jax.ShapeDtypeStruct((2 * NPAD, 16), jnp.float32),
        jax.ShapeDtypeStruct((NPAD,), jnp.float32),
    ),
    mesh=plsc.VectorSubcoreMesh(core_axis_name="c", subcore_axis_name="s"),
    scratch_types=(
        pltpu.VMEM((RPC, SUB), jnp.int32),
        pltpu.VMEM((RPC, SUB), jnp.int32),
        pltpu.VMEM((RPC, SUB), jnp.int32),
        pltpu.VMEM((RPC, SUB), jnp.int32),
        pltpu.VMEM((RPC, SUB), jnp.int32),
        pltpu.VMEM((RPC, SUB), jnp.int32),
        pltpu.VMEM((K, 16), jnp.float32),
        pltpu.VMEM((K, 16), jnp.float32),
        pltpu.VMEM((EAC + 8,), jnp.float32),
        pltpu.VMEM((EAC + 8,), jnp.float32),
        pltpu.VMEM((6, 16), jnp.float32),
        pltpu.VMEM((SUB, ), jnp.float32),
        pltpu.VMEM((ZF + 8,), jnp.float32),
        pltpu.VMEM_SHARED((NPAD, 16), jnp.float32),
        pltpu.VMEM_SHARED((NPAD,), jnp.float32),
        pltpu.SemaphoreType.DMA,
        pltpu.SemaphoreType.DMA,
        pltpu.SemaphoreType.DMA,
        pltpu.SemaphoreType.DMA,
        pltpu.SemaphoreType.DMA,
        pltpu.SemaphoreType.DMA,
    ),
    compiler_params=pltpu.CompilerParams(use_tc_tiling_on_sc=False),
)


def _node_body(x_ref, s0_ref, s1_ref, cnt_ref, bat_ref,
               W2ax_ref, W1b_ref, W2am_ref, b1b_ref, u_ref, W2au_ref,
               b2a_ref, W2b_ref, b2b_ref, out_ref):
    f32 = jnp.float32
    cnt = cnt_ref[...]
    maxc = jnp.maximum(cnt, 1.0)
    inv = 1.0 / maxc
    C = jnp.dot(W1b_ref[...], W2am_ref[...], preferred_element_type=f32)
    dvec = jnp.dot(b1b_ref[...], W2am_ref[...], preferred_element_type=f32)
    ug = jnp.dot(u_ref[...], W2au_ref[...], preferred_element_type=f32)
    Sn = jnp.concatenate([s0_ref[...], s1_ref[...]], axis=1) * inv
    z = jnp.dot(x_ref[...], W2ax_ref[...], preferred_element_type=f32)
    z = z + jnp.dot(Sn, C, preferred_element_type=f32)
    z = z + (cnt * inv) * dvec
    bat = bat_ref[...]
    iota = lax.broadcasted_iota(jnp.int32, (bat.shape[0], 256), 1)
    oh = (bat == iota).astype(f32)
    z = z + jnp.dot(oh, ug, preferred_element_type=f32)
    z = z + b2a_ref[...]
    zz = jnp.maximum(z, 0.01 * z)
    y = jnp.dot(zz, W2b_ref[...], preferred_element_type=f32) + b2b_ref[...]
    out_ref[...] = y


def kernel(x, edge_index, edge_attr, u, batch,
           W1a, b1a, W1b, b1b, W2a, b2a, W2b, b2b):
    f32 = jnp.float32
    A = W1a[:7]
    b1a2 = b1a.reshape(1, 32)

    XB_BLK = 2000
    xa0, xa1 = pl.pallas_call(
        _prep_body,
        grid=(N // XB_BLK,),
        in_specs=[
            pl.BlockSpec((XB_BLK, 7), lambda i: (i, 0)),
            pl.BlockSpec((7, 32), lambda i: (0, 0)),
            pl.BlockSpec((1, 32), lambda i: (0, 0)),
        ],
        out_specs=[
            pl.BlockSpec((XB_BLK, 16), lambda i: (i, 0)),
            pl.BlockSpec((XB_BLK, 16), lambda i: (i, 0)),
        ],
        out_shape=[
            jax.ShapeDtypeStruct((N, 16), f32),
            jax.ShapeDtypeStruct((N, 16), f32),
        ],
    )(x, A, b1a2)

    rc2 = edge_index.reshape(2 * E // SUB, SUB)
    ea_lin = edge_attr.reshape(E * 6)

    s_flat, cnt_pad = _edge_sc(rc2, ea_lin, xa0, xa1,
                               W1a[7:, :16], W1a[7:, 16:])

    x_pad = jnp.pad(x, ((0, NPAD - N), (0, 0)))
    cnt2 = cnt_pad.reshape(NPAD, 1)
    bat2 = jnp.pad(batch, (0, NPAD - N)).reshape(NPAD, 1)

    NB = RSUB
    out = pl.pallas_call(
        _node_body,
        grid=(NPAD // NB,),
        in_specs=[
            pl.BlockSpec((NB, 7), lambda i: (i, 0)),
            pl.BlockSpec((NB, 16), lambda i: (i, 0)),
            pl.BlockSpec((NB, 16), lambda i: (i + NPAD // NB, 0)),
            pl.BlockSpec((NB, 1), lambda i: (i, 0)),
            pl.BlockSpec((NB, 1), lambda i: (i, 0)),
            pl.BlockSpec((7, 32), lambda i: (0, 0)),
            pl.BlockSpec((32, 32), lambda i: (0, 0)),
            pl.BlockSpec((32, 32), lambda i: (0, 0)),
            pl.BlockSpec((1, 32), lambda i: (0, 0)),
            pl.BlockSpec((256, 64), lambda i: (0, 0)),
            pl.BlockSpec((64, 32), lambda i: (0, 0)),
            pl.BlockSpec((1, 32), lambda i: (0, 0)),
            pl.BlockSpec((32, 7), lambda i: (0, 0)),
            pl.BlockSpec((1, 7), lambda i: (0, 0)),
        ],
        out_specs=pl.BlockSpec((NB, 7), lambda i: (i, 0)),
        out_shape=jax.ShapeDtypeStruct((NPAD, 7), f32),
    )(x_pad, s_flat, s_flat, cnt2, bat2,
      W2a[:7], W1b, W2a[7:39], b1b.reshape(1, 32), u, W2a[39:],
      b2a.reshape(1, 32), W2b, b2b.reshape(1, 7))
    return out[:N]


# speedup vs baseline: 1.3115x; 1.2268x over previous
import jax
import jax.numpy as jnp
from jax import lax
from jax.experimental import pallas as pl
from jax.experimental.pallas import tpu as pltpu
from jax.experimental.pallas import tpu_sc as plsc

N = 100000
E = 3200000
NS = 16
L = 16

SUB = 80
K = 320
EAS = 8
EPW = E // NS
NCHUNK = EPW // K
RPC = K // SUB
NPAD = 100096
RSUB = NPAD // NS
ZF = RSUB // 2
EAS6 = 6
EAC = K * EAS6
NZA = RSUB // K


def _prep_body(x_ref, A_ref, b1a_ref, xa0_ref, xa1_ref):
    n = jnp.dot(x_ref[...], A_ref[...], preferred_element_type=jnp.float32)
    n = n + b1a_ref[...]
    xa0_ref[...] = n[:, :16]
    xa1_ref[...] = n[:, 16:]


def _sc_run(s, rc2, ea_hbm, xa_hbm, B_hbm,
            row_v, col_v, gath, ea_v, B_v, ones_v,
            acc_sp, cnt_sp, lsem, gsem, ssem, with_counts):
    pltpu.sync_copy(B_hbm, B_v)
    b = [B_v[k] for k in range(6)]
    splat_lo = [jnp.full((L,), k, jnp.int32) for k in range(6)]
    splat_hi = [jnp.full((L,), EAS6 + k, jnp.int32) for k in range(6)]

    def lin_issue(ci, p2, p4):
        rbase = s * (EPW // SUB) + ci * RPC
        abase = s * (EPW * EAS6) + ci * EAC
        pltpu.async_copy(rc2.at[pl.ds(rbase, RPC)], row_v[p2], lsem[p2])
        pltpu.async_copy(rc2.at[pl.ds(E // SUB + rbase, RPC)], col_v[p4],
                         lsem[p2])
        pltpu.async_copy(ea_hbm.at[pl.ds(abase, EAC)],
                         ea_v[p2].at[pl.ds(0, EAC)], lsem[p2])

    def lin_wait(p2, p4):
        pltpu.make_async_copy(rc2.at[pl.ds(0, RPC)], row_v[p2], lsem[p2]).wait()
        pltpu.make_async_copy(rc2.at[pl.ds(0, RPC)], col_v[p4], lsem[p2]).wait()
        pltpu.make_async_copy(ea_hbm.at[pl.ds(0, EAC)],
                              ea_v[p2].at[pl.ds(0, EAC)], lsem[p2]).wait()

    def gather_issue(p2):
        for j in range(RPC):
            pltpu.async_copy(xa_hbm.at[row_v[p2].at[j]],
                             gath[p2].at[pl.ds(j * SUB, SUB)], gsem[p2])

    def gather_wait(p2):
        for j in range(RPC):
            pltpu.make_async_copy(xa_hbm.at[row_v[p2].at[j]],
                                  gath[p2].at[pl.ds(j * SUB, SUB)],
                                  gsem[p2]).wait()

    def scatter_issue(p2, p4):
        for j in range(RPC):
            pltpu.async_copy(gath[p2].at[pl.ds(j * SUB, SUB)],
                             acc_sp.at[col_v[p4].at[j]], ssem[p2], add=True)
        if with_counts:
            for j in range(RPC):
                pltpu.async_copy(ones_v.at[pl.ds(0, SUB)],
                                 cnt_sp.at[col_v[p4].at[j]], ssem[p2], add=True)

    def scatter_drain(p2, p4):
        for j in range(RPC):
            pltpu.make_async_copy(gath[p2].at[pl.ds(j * SUB, SUB)],
                                  acc_sp.at[col_v[p4].at[j]], ssem[p2]).wait()
        if with_counts:
            for j in range(RPC):
                pltpu.make_async_copy(ones_v.at[pl.ds(0, SUB)],
                                      cnt_sp.at[col_v[p4].at[j]], ssem[p2]).wait()

    def compute(p2):
        gv = gath[p2]
        ev = ea_v[p2]

        @plsc.parallel_loop(0, K // 2, step=1, unroll=4)
        def body(p):
            r = p * 2
            ea_vec = ev[pl.ds(p * 2 * EAS6, L)]
            v = gv[r]
            w = gv[r + 1]
            for k in range(6):
                ek = ea_vec.at[splat_lo[k]].get(mode="promise_in_bounds")
                fk = ea_vec.at[splat_hi[k]].get(mode="promise_in_bounds")
                v = v + ek * b[k]
                w = w + fk * b[k]
            gv[r] = jnp.maximum(v, 0.01 * v)
            gv[r + 1] = jnp.maximum(w, 0.01 * w)

    lin_issue(0, 0, 0)
    lin_issue(1, 1, 1)
    lin_wait(0, 0)
    gather_issue(0)

    def macro(I, carry):
        for p in range(4):
            i = I * 4 + p
            s2 = p % 2
            s2n = (p + 1) % 2
            s4 = p
            s4n = (p + 1) % 4
            s4nn = (p + 2) % 4
            if p == 0:
                @pl.when(I >= 1)
                def _():
                    scatter_drain(1, 3)
            else:
                scatter_drain(s2n, s4 - 1)
            @pl.when(i <= NCHUNK - 2)
            def _():
                lin_wait(s2n, s4n)
                gather_issue(s2n)
            gather_wait(s2)
            compute(s2)
            scatter_issue(s2, s4)
            @pl.when(i <= NCHUNK - 3)
            def _():
                lin_issue(i + 2, s2, s4nn)
        return carry

    lax.fori_loop(0, NCHUNK // 4, macro, 0)
    scatter_drain(1, 3)


def _edge_sc_body(rc2, ea_flat, xa0, xa1, B0, B1,
                  s_out, cnt_out,
                  row_v0, row_v1, col_v0, col_v1, col_v2, col_v3,
                  gath0, gath1, ea_v0, ea_v1, B_v, ones_v, zflat,
                  acc_sp, cnt_sp,
                  lsem0, lsem1, gsem0, gsem1, ssem0, ssem1):
    c = lax.axis_index("c")
    s = lax.axis_index("s")
    row_v = [row_v0, row_v1]
    col_v = [col_v0, col_v1, col_v2, col_v3]
    gath = [gath0, gath1]
    ea_v = [ea_v0, ea_v1]
    lsem = [lsem0, lsem1]
    gsem = [gsem0, gsem1]
    ssem = [ssem0, ssem1]

    zero16 = jnp.zeros((L,), jnp.float32)
    one16 = jnp.ones((L,), jnp.float32)

    def zr(i, cc):
        gath0[i] = zero16
        return cc
    lax.fori_loop(0, K, zr, 0)

    def zf(i, cc):
        zflat[pl.ds(i * L, L)] = zero16
        return cc
    lax.fori_loop(0, (ZF + 8) // L, zf, 0)
    for j in range(SUB // L):
        ones_v[pl.ds(j * L, L)] = one16

    rb = s * RSUB
    for j in range(NZA):
        pltpu.sync_copy(gath0, acc_sp.at[pl.ds(rb + j * K, K)])
    pltpu.sync_copy(gath0.at[pl.ds(0, RSUB - NZA * K)],
                    acc_sp.at[pl.ds(rb + NZA * K, RSUB - NZA * K)])
    for j in range(2):
        pltpu.sync_copy(zflat.at[pl.ds(0, ZF)], cnt_sp.at[pl.ds(rb + j * ZF, ZF)])

    plsc.subcore_barrier()

    @pl.when(c == 0)
    def _():
        _sc_run(s, rc2, ea_flat, xa0, B0, row_v, col_v, gath, ea_v,
                B_v, ones_v, acc_sp, cnt_sp, lsem, gsem, ssem,
                with_counts=True)

    @pl.when(c == 1)
    def _():
        _sc_run(s, rc2, ea_flat, xa1, B1, row_v, col_v, gath, ea_v,
                B_v, ones_v, acc_sp, cnt_sp, lsem, gsem, ssem,
                with_counts=False)

    plsc.subcore_barrier()

    pltpu.sync_copy(acc_sp.at[pl.ds(rb, RSUB)],
                    s_out.at[pl.ds(c * NPAD + rb, RSUB)])

    @pl.when(c == 0)
    def _():
        pltpu.sync_copy(cnt_sp.at[pl.ds(rb, RSUB)], cnt_out.at[pl.ds(rb, RSUB)])


_edge_sc = pl.kernel(
    _edge_sc_body,
    out_type=(
        jax.ShapeDtypeStruct((2 * NPAD, 16), jnp.float32),
        jax.ShapeDtypeStruct((NPAD,), jnp.float32),
    ),
    mesh=plsc.VectorSubcoreMesh(core_axis_name="c", subcore_axis_name="s"),
    scratch_types=(
        pltpu.VMEM((RPC, SUB), jnp.int32),
        pltpu.VMEM((RPC, SUB), jnp.int32),
        pltpu.VMEM((RPC, SUB), jnp.int32),
        pltpu.VMEM((RPC, SUB), jnp.int32),
        pltpu.VMEM((RPC, SUB), jnp.int32),
        pltpu.VMEM((RPC, SUB), jnp.int32),
        pltpu.VMEM((K, 16), jnp.float32),
        pltpu.VMEM((K, 16), jnp.float32),
        pltpu.VMEM((EAC + 8,), jnp.float32),
        pltpu.VMEM((EAC + 8,), jnp.float32),
        pltpu.VMEM((6, 16), jnp.float32),
        pltpu.VMEM((SUB, ), jnp.float32),
        pltpu.VMEM((ZF + 8,), jnp.float32),
        pltpu.VMEM_SHARED((NPAD, 16), jnp.float32),
        pltpu.VMEM_SHARED((NPAD,), jnp.float32),
        pltpu.SemaphoreType.DMA,
        pltpu.SemaphoreType.DMA,
        pltpu.SemaphoreType.DMA,
        pltpu.SemaphoreType.DMA,
        pltpu.SemaphoreType.DMA,
        pltpu.SemaphoreType.DMA,
    ),
    compiler_params=pltpu.CompilerParams(use_tc_tiling_on_sc=False),
)


def _node_body(x_ref, s0_ref, s1_ref, cnt_ref, bat_ref,
               W2ax_ref, W1b_ref, W2am_ref, b1b_ref, u_ref, W2au_ref,
               b2a_ref, W2b_ref, b2b_ref, out_ref):
    f32 = jnp.float32
    cnt = cnt_ref[...]
    maxc = jnp.maximum(cnt, 1.0)
    inv = 1.0 / maxc
    C = jnp.dot(W1b_ref[...], W2am_ref[...], preferred_element_type=f32)
    dvec = jnp.dot(b1b_ref[...], W2am_ref[...], preferred_element_type=f32)
    ug = jnp.dot(u_ref[...], W2au_ref[...], preferred_element_type=f32)
    Sn = jnp.concatenate([s0_ref[...], s1_ref[...]], axis=1) * inv
    z = jnp.dot(x_ref[...], W2ax_ref[...], preferred_element_type=f32)
    z = z + jnp.dot(Sn, C, preferred_element_type=f32)
    z = z + (cnt * inv) * dvec
    bat = bat_ref[...]
    iota = lax.broadcasted_iota(jnp.int32, (bat.shape[0], 256), 1)
    oh = (bat == iota).astype(f32)
    z = z + jnp.dot(oh, ug, preferred_element_type=f32)
    z = z + b2a_ref[...]
    zz = jnp.maximum(z, 0.01 * z)
    y = jnp.dot(zz, W2b_ref[...], preferred_element_type=f32) + b2b_ref[...]
    out_ref[...] = y


def kernel(x, edge_index, edge_attr, u, batch,
           W1a, b1a, W1b, b1b, W2a, b2a, W2b, b2b):
    f32 = jnp.float32
    A = W1a[:7]
    b1a2 = b1a.reshape(1, 32)

    XB_BLK = 2000
    xa0, xa1 = pl.pallas_call(
        _prep_body,
        grid=(N // XB_BLK,),
        in_specs=[
            pl.BlockSpec((XB_BLK, 7), lambda i: (i, 0)),
            pl.BlockSpec((7, 32), lambda i: (0, 0)),
            pl.BlockSpec((1, 32), lambda i: (0, 0)),
        ],
        out_specs=[
            pl.BlockSpec((XB_BLK, 16), lambda i: (i, 0)),
            pl.BlockSpec((XB_BLK, 16), lambda i: (i, 0)),
        ],
        out_shape=[
            jax.ShapeDtypeStruct((N, 16), f32),
            jax.ShapeDtypeStruct((N, 16), f32),
        ],
    )(x, A, b1a2)

    rc2 = edge_index.reshape(2 * E // SUB, SUB)
    ea_lin = edge_attr.reshape(E * 6)

    s_flat, cnt_pad = _edge_sc(rc2, ea_lin, xa0, xa1,
                               W1a[7:, :16], W1a[7:, 16:])

    x_pad = jnp.pad(x, ((0, NPAD - N), (0, 0)))
    cnt2 = cnt_pad.reshape(NPAD, 1)
    bat2 = jnp.pad(batch, (0, NPAD - N)).reshape(NPAD, 1)

    NB = RSUB
    out = pl.pallas_call(
        _node_body,
        grid=(NPAD // NB,),
        in_specs=[
            pl.BlockSpec((NB, 7), lambda i: (i, 0)),
            pl.BlockSpec((NB, 16), lambda i: (i, 0)),
            pl.BlockSpec((NB, 16), lambda i: (i + NPAD // NB, 0)),
            pl.BlockSpec((NB, 1), lambda i: (i, 0)),
            pl.BlockSpec((NB, 1), lambda i: (i, 0)),
            pl.BlockSpec((7, 32), lambda i: (0, 0)),
            pl.BlockSpec((32, 32), lambda i: (0, 0)),
            pl.BlockSpec((32, 32), lambda i: (0, 0)),
            pl.BlockSpec((1, 32), lambda i: (0, 0)),
            pl.BlockSpec((256, 64), lambda i: (0, 0)),
            pl.BlockSpec((64, 32), lambda i: (0, 0)),
            pl.BlockSpec((1, 32), lambda i: (0, 0)),
            pl.BlockSpec((32, 7), lambda i: (0, 0)),
            pl.BlockSpec((1, 7), lambda i: (0, 0)),
        ],
        out_specs=pl.BlockSpec((NB, 7), lambda i: (i, 0)),
        out_shape=jax.ShapeDtypeStruct((NPAD, 7), f32),
    )(x_pad, s_flat, s_flat, cnt2, bat2,
      W2a[:7], W1b, W2a[7:39], b1b.reshape(1, 32), u, W2a[39:],
      b2a.reshape(1, 32), W2b, b2b.reshape(1, 7))
    return out[:N]


# speedup vs baseline: 1.3183x; 1.0053x over previous
import jax
import jax.numpy as jnp
from jax import lax
from jax.experimental import pallas as pl
from jax.experimental.pallas import tpu as pltpu
from jax.experimental.pallas import tpu_sc as plsc

N = 100000
E = 3200000
NS = 16
L = 16

SUB = 80
K = 320
EAS = 8
EPW = E // NS
NCHUNK = EPW // K
RPC = K // SUB
NPAD = 100096
RSUB = NPAD // NS
ZF = RSUB // 2
EAS6 = 6
EAC = K * EAS6
NZA = RSUB // K


def _prep_body(x_ref, A_ref, b1a_ref, xa0_ref, xa1_ref):
    n = jnp.dot(x_ref[...], A_ref[...], preferred_element_type=jnp.float32)
    n = n + b1a_ref[...]
    xa0_ref[...] = n[:, :16]
    xa1_ref[...] = n[:, 16:]


def _sc_run(s, rc2, ea_hbm, xa_hbm, B_hbm,
            row_v, col_v, gath, ea_v, B_v, ones_v,
            acc_sp, cnt_sp, lsem, gsem, ssem, with_counts):
    pltpu.sync_copy(B_hbm, B_v)
    b = [B_v[k] for k in range(6)]
    splat_lo = [jnp.full((L,), k, jnp.int32) for k in range(6)]
    splat_hi = [jnp.full((L,), EAS6 + k, jnp.int32) for k in range(6)]

    def lin_issue(ci, p2, p4):
        rbase = s * (EPW // SUB) + ci * RPC
        abase = s * (EPW * EAS6) + ci * EAC
        pltpu.async_copy(rc2.at[pl.ds(rbase, RPC)], row_v[p2], lsem[p2])
        pltpu.async_copy(rc2.at[pl.ds(E // SUB + rbase, RPC)], col_v[p4],
                         lsem[p2])
        pltpu.async_copy(ea_hbm.at[pl.ds(abase, EAC)],
                         ea_v[p2].at[pl.ds(0, EAC)], lsem[p2])

    def lin_wait(p2, p4):
        pltpu.make_async_copy(rc2.at[pl.ds(0, RPC)], row_v[p2], lsem[p2]).wait()
        pltpu.make_async_copy(rc2.at[pl.ds(0, RPC)], col_v[p4], lsem[p2]).wait()
        pltpu.make_async_copy(ea_hbm.at[pl.ds(0, EAC)],
                              ea_v[p2].at[pl.ds(0, EAC)], lsem[p2]).wait()

    def gather_issue(p2):
        for j in range(RPC):
            pltpu.async_copy(xa_hbm.at[row_v[p2].at[j]],
                             gath[p2].at[pl.ds(j * SUB, SUB)], gsem[p2])

    def gather_wait(p2):
        for j in range(RPC):
            pltpu.make_async_copy(xa_hbm.at[row_v[p2].at[j]],
                                  gath[p2].at[pl.ds(j * SUB, SUB)],
                                  gsem[p2]).wait()

    def scatter_issue(p2, p4):
        for j in range(RPC):
            pltpu.async_copy(gath[p2].at[pl.ds(j * SUB, SUB)],
                             acc_sp.at[col_v[p4].at[j]], ssem[p2], add=True)
        if with_counts:
            for j in range(RPC):
                pltpu.async_copy(ones_v.at[pl.ds(0, SUB)],
                                 cnt_sp.at[col_v[p4].at[j]], ssem[p2], add=True)

    def scatter_drain(p2, p4):
        for j in range(RPC):
            pltpu.make_async_copy(gath[p2].at[pl.ds(j * SUB, SUB)],
                                  acc_sp.at[col_v[p4].at[j]], ssem[p2]).wait()
        if with_counts:
            for j in range(RPC):
                pltpu.make_async_copy(ones_v.at[pl.ds(0, SUB)],
                                      cnt_sp.at[col_v[p4].at[j]], ssem[p2]).wait()

    def compute(p2):
        gv = gath[p2]
        ev = ea_v[p2]

        @plsc.parallel_loop(0, K // 2, step=1, unroll=8)
        def body(p):
            r = p * 2
            ea_vec = ev[pl.ds(p * 2 * EAS6, L)]
            v = gv[r]
            w = gv[r + 1]
            for k in range(6):
                ek = ea_vec.at[splat_lo[k]].get(mode="promise_in_bounds")
                fk = ea_vec.at[splat_hi[k]].get(mode="promise_in_bounds")
                v = v + ek * b[k]
                w = w + fk * b[k]
            gv[r] = jnp.maximum(v, 0.01 * v)
            gv[r + 1] = jnp.maximum(w, 0.01 * w)

    lin_issue(0, 0, 0)
    lin_issue(1, 1, 1)
    lin_wait(0, 0)
    gather_issue(0)

    def macro(I, carry):
        for p in range(4):
            i = I * 4 + p
            s2 = p % 2
            s2n = (p + 1) % 2
            s4 = p
            s4n = (p + 1) % 4
            s4nn = (p + 2) % 4
            if p == 0:
                @pl.when(I >= 1)
                def _():
                    scatter_drain(1, 3)
            else:
                scatter_drain(s2n, s4 - 1)
            @pl.when(i <= NCHUNK - 2)
            def _():
                lin_wait(s2n, s4n)
                gather_issue(s2n)
            gather_wait(s2)
            compute(s2)
            scatter_issue(s2, s4)
            @pl.when(i <= NCHUNK - 3)
            def _():
                lin_issue(i + 2, s2, s4nn)
        return carry

    lax.fori_loop(0, NCHUNK // 4, macro, 0)
    scatter_drain(1, 3)


def _edge_sc_body(rc2, ea_flat, xa0, xa1, B0, B1,
                  s_out, cnt_out,
                  row_v0, row_v1, col_v0, col_v1, col_v2, col_v3,
                  gath0, gath1, ea_v0, ea_v1, B_v, ones_v, zflat,
                  acc_sp, cnt_sp,
                  lsem0, lsem1, gsem0, gsem1, ssem0, ssem1):
    c = lax.axis_index("c")
    s = lax.axis_index("s")
    row_v = [row_v0, row_v1]
    col_v = [col_v0, col_v1, col_v2, col_v3]
    gath = [gath0, gath1]
    ea_v = [ea_v0, ea_v1]
    lsem = [lsem0, lsem1]
    gsem = [gsem0, gsem1]
    ssem = [ssem0, ssem1]

    zero16 = jnp.zeros((L,), jnp.float32)
    one16 = jnp.ones((L,), jnp.float32)

    def zr(i, cc):
        gath0[i] = zero16
        return cc
    lax.fori_loop(0, K, zr, 0)

    def zf(i, cc):
        zflat[pl.ds(i * L, L)] = zero16
        return cc
    lax.fori_loop(0, (ZF + 8) // L, zf, 0)
    for j in range(SUB // L):
        ones_v[pl.ds(j * L, L)] = one16

    rb = s * RSUB
    for j in range(NZA):
        pltpu.sync_copy(gath0, acc_sp.at[pl.ds(rb + j * K, K)])
    pltpu.sync_copy(gath0.at[pl.ds(0, RSUB - NZA * K)],
                    acc_sp.at[pl.ds(rb + NZA * K, RSUB - NZA * K)])
    for j in range(2):
        pltpu.sync_copy(zflat.at[pl.ds(0, ZF)], cnt_sp.at[pl.ds(rb + j * ZF, ZF)])

    plsc.subcore_barrier()

    @pl.when(c == 0)
    def _():
        _sc_run(s, rc2, ea_flat, xa0, B0, row_v, col_v, gath, ea_v,
                B_v, ones_v, acc_sp, cnt_sp, lsem, gsem, ssem,
                with_counts=True)

    @pl.when(c == 1)
    def _():
        _sc_run(s, rc2, ea_flat, xa1, B1, row_v, col_v, gath, ea_v,
                B_v, ones_v, acc_sp, cnt_sp, lsem, gsem, ssem,
                with_counts=False)

    plsc.subcore_barrier()

    pltpu.sync_copy(acc_sp.at[pl.ds(rb, RSUB)],
                    s_out.at[pl.ds(c * NPAD + rb, RSUB)])

    @pl.when(c == 0)
    def _():
        pltpu.sync_copy(cnt_sp.at[pl.ds(rb, RSUB)], cnt_out.at[pl.ds(rb, RSUB)])


_edge_sc = pl.kernel(
    _edge_sc_body,
    out_type=(
        jax.ShapeDtypeStruct((2 * NPAD, 16), jnp.float32),
        jax.ShapeDtypeStruct((NPAD,), jnp.float32),
    ),
    mesh=plsc.VectorSubcoreMesh(core_axis_name="c", subcore_axis_name="s"),
    scratch_types=(
        pltpu.VMEM((RPC, SUB), jnp.int32),
        pltpu.VMEM((RPC, SUB), jnp.int32),
        pltpu.VMEM((RPC, SUB), jnp.int32),
        pltpu.VMEM((RPC, SUB), jnp.int32),
        pltpu.VMEM((RPC, SUB), jnp.int32),
        pltpu.VMEM((RPC, SUB), jnp.int32),
        pltpu.VMEM((K, 16), jnp.float32),
        pltpu.VMEM((K, 16), jnp.float32),
        pltpu.VMEM((EAC + 8,), jnp.float32),
        pltpu.VMEM((EAC + 8,), jnp.float32),
        pltpu.VMEM((6, 16), jnp.float32),
        pltpu.VMEM((SUB, ), jnp.float32),
        pltpu.VMEM((ZF + 8,), jnp.float32),
        pltpu.VMEM_SHARED((NPAD, 16), jnp.float32),
        pltpu.VMEM_SHARED((NPAD,), jnp.float32),
        pltpu.SemaphoreType.DMA,
        pltpu.SemaphoreType.DMA,
        pltpu.SemaphoreType.DMA,
        pltpu.SemaphoreType.DMA,
        pltpu.SemaphoreType.DMA,
        pltpu.SemaphoreType.DMA,
    ),
    compiler_params=pltpu.CompilerParams(use_tc_tiling_on_sc=False),
)


def _node_body(x_ref, s0_ref, s1_ref, cnt_ref, bat_ref,
               W2ax_ref, W1b_ref, W2am_ref, b1b_ref, u_ref, W2au_ref,
               b2a_ref, W2b_ref, b2b_ref, out_ref):
    f32 = jnp.float32
    cnt = cnt_ref[...]
    maxc = jnp.maximum(cnt, 1.0)
    inv = 1.0 / maxc
    C = jnp.dot(W1b_ref[...], W2am_ref[...], preferred_element_type=f32)
    dvec = jnp.dot(b1b_ref[...], W2am_ref[...], preferred_element_type=f32)
    ug = jnp.dot(u_ref[...], W2au_ref[...], preferred_element_type=f32)
    Sn = jnp.concatenate([s0_ref[...], s1_ref[...]], axis=1) * inv
    z = jnp.dot(x_ref[...], W2ax_ref[...], preferred_element_type=f32)
    z = z + jnp.dot(Sn, C, preferred_element_type=f32)
    z = z + (cnt * inv) * dvec
    bat = bat_ref[...]
    iota = lax.broadcasted_iota(jnp.int32, (bat.shape[0], 256), 1)
    oh = (bat == iota).astype(f32)
    z = z + jnp.dot(oh, ug, preferred_element_type=f32)
    z = z + b2a_ref[...]
    zz = jnp.maximum(z, 0.01 * z)
    y = jnp.dot(zz, W2b_ref[...], preferred_element_type=f32) + b2b_ref[...]
    out_ref[...] = y


def kernel(x, edge_index, edge_attr, u, batch,
           W1a, b1a, W1b, b1b, W2a, b2a, W2b, b2b):
    f32 = jnp.float32
    A = W1a[:7]
    b1a2 = b1a.reshape(1, 32)

    XB_BLK = 2000
    xa0, xa1 = pl.pallas_call(
        _prep_body,
        grid=(N // XB_BLK,),
        in_specs=[
            pl.BlockSpec((XB_BLK, 7), lambda i: (i, 0)),
            pl.BlockSpec((7, 32), lambda i: (0, 0)),
            pl.BlockSpec((1, 32), lambda i: (0, 0)),
        ],
        out_specs=[
            pl.BlockSpec((XB_BLK, 16), lambda i: (i, 0)),
            pl.BlockSpec((XB_BLK, 16), lambda i: (i, 0)),
        ],
        out_shape=[
            jax.ShapeDtypeStruct((N, 16), f32),
            jax.ShapeDtypeStruct((N, 16), f32),
        ],
    )(x, A, b1a2)

    rc2 = edge_index.reshape(2 * E // SUB, SUB)
    ea_lin = edge_attr.reshape(E * 6)

    s_flat, cnt_pad = _edge_sc(rc2, ea_lin, xa0, xa1,
                               W1a[7:, :16], W1a[7:, 16:])

    x_pad = jnp.pad(x, ((0, NPAD - N), (0, 0)))
    cnt2 = cnt_pad.reshape(NPAD, 1)
    bat2 = jnp.pad(batch, (0, NPAD - N)).reshape(NPAD, 1)

    NB = RSUB
    out = pl.pallas_call(
        _node_body,
        grid=(NPAD // NB,),
        in_specs=[
            pl.BlockSpec((NB, 7), lambda i: (i, 0)),
            pl.BlockSpec((NB, 16), lambda i: (i, 0)),
            pl.BlockSpec((NB, 16), lambda i: (i + NPAD // NB, 0)),
            pl.BlockSpec((NB, 1), lambda i: (i, 0)),
            pl.BlockSpec((NB, 1), lambda i: (i, 0)),
            pl.BlockSpec((7, 32), lambda i: (0, 0)),
            pl.BlockSpec((32, 32), lambda i: (0, 0)),
            pl.BlockSpec((32, 32), lambda i: (0, 0)),
            pl.BlockSpec((1, 32), lambda i: (0, 0)),
            pl.BlockSpec((256, 64), lambda i: (0, 0)),
            pl.BlockSpec((64, 32), lambda i: (0, 0)),
            pl.BlockSpec((1, 32), lambda i: (0, 0)),
            pl.BlockSpec((32, 7), lambda i: (0, 0)),
            pl.BlockSpec((1, 7), lambda i: (0, 0)),
        ],
        out_specs=pl.BlockSpec((NB, 7), lambda i: (i, 0)),
        out_shape=jax.ShapeDtypeStruct((NPAD, 7), f32),
    )(x_pad, s_flat, s_flat, cnt2, bat2,
      W2a[:7], W1b, W2a[7:39], b1b.reshape(1, 32), u, W2a[39:],
      b2a.reshape(1, 32), W2b, b2b.reshape(1, 7))
    return out[:N]
